# Initial kernel scaffold; baseline (speedup 1.0000x reference)
import jax
import jax.numpy as jnp
from jax.experimental import pallas as pl


def kernel(x, f_ij, idx_i, idx_j, rcut_ij, W_in2f, Wf1, bf1, Wf2, bf2, Wo1, bo1, Wo2, bo2):
    raise NotImplementedError("write your pallas kernel here")


# speedup vs baseline: 1.4027x; 1.4027x over previous
import functools

import jax
import jax.numpy as jnp
from jax import lax
from jax.experimental import pallas as pl
from jax.experimental.pallas import tpu as pltpu
from jax.experimental.pallas import tpu_sc as plsc

N = 10000
E = 160000
D = 256
R = 16
H = 128

CH = 128
NCHUNK = 79
EPT = NCHUNK * CH
E_PAD = 16 * EPT
NPAD = 10240
ROWS_PER_TEC = NPAD // 16

_LOG2 = 0.6931471805599453


def _ssp(t):
    return jnp.maximum(t, 0.0) + jnp.log1p(jnp.exp(-jnp.abs(t))) - _LOG2


def _in2f_body(x_ref, w_ref, lo_ref, hi_ref):
    h = jnp.dot(x_ref[...], w_ref[...], preferred_element_type=jnp.float32)
    lo_ref[...] = h[:, :H]
    hi_ref[...] = h[:, H:]


def _filter_body(f_ref, rc_ref, wf1_ref, bf1_ref, wf2_ref, bf2_ref,
                 lo_ref, hi_ref):
    g = jnp.dot(f_ref[...], wf1_ref[...], preferred_element_type=jnp.float32)
    g = _ssp(g + bf1_ref[...])
    w = jnp.dot(g, wf2_ref[...], preferred_element_type=jnp.float32)
    w = (w + bf2_ref[...]) * rc_ref[...]
    lo_ref[...] = w[:, :H]
    hi_ref[...] = w[:, H:]


def _out_body(lo_ref, hi_ref, wo1_ref, bo1_ref, wo2_ref, bo2_ref, o_ref):
    a = jnp.dot(lo_ref[...], wo1_ref[:H, :], preferred_element_type=jnp.float32)
    a = a + jnp.dot(hi_ref[...], wo1_ref[H:, :],
                    preferred_element_type=jnp.float32)
    a = _ssp(a + bo1_ref[...])
    o_ref[...] = jnp.dot(a, wo2_ref[...],
                         preferred_element_type=jnp.float32) + bo2_ref[...]


def _cfconv_sc_body(hlo, hhi, wlo, whi, idxj_hbm, idxi_hbm,
                    out_lo, out_hi,
                    idxj_v, idxi_v, rows_v, wij_v, agg_s, sem):
    c = lax.axis_index("c")
    s = lax.axis_index("s")

    z16 = jnp.zeros((16,), jnp.float32)

    def _zero_row(r, carry):
        for j in range(8):
            rows_v[r, pl.ds(j * 16, 16)] = z16
        return carry
    lax.fori_loop(0, CH, _zero_row, 0)

    r_base = s * ROWS_PER_TEC
    for k in range(ROWS_PER_TEC // CH):
        pltpu.sync_copy(rows_v, agg_s.at[pl.ds(r_base + k * CH, CH), :])

    plsc.subcore_barrier()

    def _run(h_ref, w_ref):
        e_base = s * EPT

        def _chunk(ci, carry):
            e0 = e_base + ci * CH
            pltpu.sync_copy(idxj_hbm.at[pl.ds(e0, CH)], idxj_v)
            pltpu.sync_copy(idxi_hbm.at[pl.ds(e0, CH)], idxi_v)
            pltpu.async_copy(h_ref.at[idxj_v], rows_v, sem).wait()
            pltpu.sync_copy(w_ref.at[pl.ds(e0, CH), :], wij_v)

            def _mul(r, cc):
                for j in range(8):
                    sl = pl.ds(j * 16, 16)
                    rows_v[r, sl] = rows_v[r, sl] * wij_v[r, sl]
                return cc
            lax.fori_loop(0, CH, _mul, 0)

            pltpu.sync_copy(rows_v, agg_s.at[idxi_v], add=True)
            return carry
        lax.fori_loop(0, NCHUNK, _chunk, 0)

    @pl.when(c == 0)
    def _():
        _run(hlo, wlo)

    @pl.when(c == 1)
    def _():
        _run(hhi, whi)

    plsc.subcore_barrier()

    for k in range(ROWS_PER_TEC // CH):
        rr = r_base + k * CH
        pltpu.sync_copy(agg_s.at[pl.ds(rr, CH), :], wij_v)

        @pl.when(c == 0)
        def _():
            pltpu.sync_copy(wij_v, out_lo.at[pl.ds(rr, CH), :])

        @pl.when(c == 1)
        def _():
            pltpu.sync_copy(wij_v, out_hi.at[pl.ds(rr, CH), :])


@functools.cache
def _make_cfconv_sc():
    return functools.partial(
        pl.kernel,
        mesh=plsc.VectorSubcoreMesh(core_axis_name="c", subcore_axis_name="s"),
        out_type=(jax.ShapeDtypeStruct((NPAD, H), jnp.float32),
                  jax.ShapeDtypeStruct((NPAD, H), jnp.float32)),
        scratch_types=[
            pltpu.VMEM((CH,), jnp.int32),
            pltpu.VMEM((CH,), jnp.int32),
            pltpu.VMEM((CH, H), jnp.float32),
            pltpu.VMEM((CH, H), jnp.float32),
            pltpu.VMEM_SHARED((NPAD, H), jnp.float32),
            pltpu.SemaphoreType.DMA,
        ],
    )(_cfconv_sc_body)


def kernel(x, f_ij, idx_i, idx_j, rcut_ij,
           W_in2f, Wf1, bf1, Wf2, bf2, Wo1, bo1, Wo2, bo2):
    h_lo, h_hi = pl.pallas_call(
        _in2f_body,
        grid=(10,),
        in_specs=[
            pl.BlockSpec((N // 10, D), lambda i: (i, 0)),
            pl.BlockSpec((D, D), lambda i: (0, 0)),
        ],
        out_specs=[
            pl.BlockSpec((N // 10, H), lambda i: (i, 0)),
            pl.BlockSpec((N // 10, H), lambda i: (i, 0)),
        ],
        out_shape=[
            jax.ShapeDtypeStruct((N, H), jnp.float32),
            jax.ShapeDtypeStruct((N, H), jnp.float32),
        ],
    )(x, W_in2f)

    pad = E_PAD - E
    f_pad = jnp.pad(f_ij, ((0, pad), (0, 0)))
    rc_pad = jnp.pad(rcut_ij, (0, pad))[:, None]
    idxj32 = jnp.pad(idx_j.astype(jnp.int32), (0, pad))
    idxi32 = jnp.pad(idx_i.astype(jnp.int32), (0, pad))

    BE = 512
    w_lo, w_hi = pl.pallas_call(
        _filter_body,
        grid=(E_PAD // BE,),
        in_specs=[
            pl.BlockSpec((BE, R), lambda i: (i, 0)),
            pl.BlockSpec((BE, 1), lambda i: (i, 0)),
            pl.BlockSpec((R, D), lambda i: (0, 0)),
            pl.BlockSpec((1, D), lambda i: (0, 0)),
            pl.BlockSpec((D, D), lambda i: (0, 0)),
            pl.BlockSpec((1, D), lambda i: (0, 0)),
        ],
        out_specs=[
            pl.BlockSpec((BE, H), lambda i: (i, 0)),
            pl.BlockSpec((BE, H), lambda i: (i, 0)),
        ],
        out_shape=[
            jax.ShapeDtypeStruct((E_PAD, H), jnp.float32),
            jax.ShapeDtypeStruct((E_PAD, H), jnp.float32),
        ],
    )(f_pad, rc_pad, Wf1, bf1[None, :], Wf2, bf2[None, :])

    agg_lo, agg_hi = _make_cfconv_sc()(h_lo, h_hi, w_lo, w_hi, idxj32, idxi32)

    BN = 1024
    out = pl.pallas_call(
        _out_body,
        grid=(NPAD // BN,),
        in_specs=[
            pl.BlockSpec((BN, H), lambda i: (i, 0)),
            pl.BlockSpec((BN, H), lambda i: (i, 0)),
            pl.BlockSpec((D, D), lambda i: (0, 0)),
            pl.BlockSpec((1, D), lambda i: (0, 0)),
            pl.BlockSpec((D, D), lambda i: (0, 0)),
            pl.BlockSpec((1, D), lambda i: (0, 0)),
        ],
        out_specs=pl.BlockSpec((BN, D), lambda i: (i, 0)),
        out_shape=jax.ShapeDtypeStruct((NPAD, D), jnp.float32),
    )(agg_lo, agg_hi, Wo1, bo1[None, :], Wo2, bo2[None, :])

    return out[:N]


# speedup vs baseline: 1.7628x; 1.2567x over previous
import functools

import jax
import jax.numpy as jnp
from jax import lax
from jax.experimental import pallas as pl
from jax.experimental.pallas import tpu as pltpu
from jax.experimental.pallas import tpu_sc as plsc

N = 10000
E = 160000
D = 256
R = 16
H = 128

CH = 64
NCHUNK = 80
EPT = NCHUNK * CH
E_HALF = 16 * EPT
E_PAD = 2 * E_HALF
NPAD = 10240
ROWS_PER_TEC = NPAD // 16

_LOG2 = 0.6931471805599453


def _ssp(t):
    return jnp.maximum(t, 0.0) + jnp.log1p(jnp.exp(-jnp.abs(t))) - _LOG2


def _in2f_body(x_ref, w_ref, lo_ref, hi_ref):
    h = jnp.dot(x_ref[...].astype(jnp.bfloat16),
                w_ref[...].astype(jnp.bfloat16),
                preferred_element_type=jnp.float32)
    lo_ref[...] = h[:, :H]
    hi_ref[...] = h[:, H:]


def _filter_body(f_ref, rc_ref, wf1_ref, bf1_ref, wf2_ref, bf2_ref,
                 lo_ref, hi_ref):
    g = jnp.dot(f_ref[...], wf1_ref[...], preferred_element_type=jnp.float32)
    g = _ssp(g + bf1_ref[...])
    w = jnp.dot(g.astype(jnp.bfloat16), wf2_ref[...].astype(jnp.bfloat16),
                preferred_element_type=jnp.float32)
    w = (w + bf2_ref[...]) * rc_ref[...]
    lo_ref[...] = w[:, :H]
    hi_ref[...] = w[:, H:]


def _out_body(lo_a_ref, hi_a_ref, lo_b_ref, hi_b_ref,
              wo1_ref, bo1_ref, wo2_ref, bo2_ref, o_ref):
    lo = lo_a_ref[...] + lo_b_ref[...]
    hi = hi_a_ref[...] + hi_b_ref[...]
    a = jnp.dot(lo, wo1_ref[:H, :], preferred_element_type=jnp.float32)
    a = a + jnp.dot(hi, wo1_ref[H:, :], preferred_element_type=jnp.float32)
    a = _ssp(a + bo1_ref[...])
    o_ref[...] = jnp.dot(a, wo2_ref[...],
                         preferred_element_type=jnp.float32) + bo2_ref[...]


def _cfconv_sc_body(eoff, hlo, hhi, wlo, whi, idxj_hbm, idxi_hbm,
                    out_lo, out_hi,
                    idxj0, idxj1, idxi0, idxi1, rows0, rows1, wij0, wij1,
                    agg_s, sg0, sg1, sw0, sw1, sj0, sj1, sii0, sii1,
                    ssc0, ssc1):
    c = lax.axis_index("c")
    s = lax.axis_index("s")
    idxj = (idxj0, idxj1)
    idxi = (idxi0, idxi1)
    rows = (rows0, rows1)
    wij = (wij0, wij1)
    sj = (sj0, sj1)
    sii = (sii0, sii1)
    sg = (sg0, sg1)
    sw = (sw0, sw1)
    ssc = (ssc0, ssc1)

    z16 = jnp.zeros((16,), jnp.float32)

    def _zero_row(r, carry):
        for j in range(8):
            rows0[r, pl.ds(j * 16, 16)] = z16
        return carry
    lax.fori_loop(0, CH, _zero_row, 0)

    r_base = s * ROWS_PER_TEC
    for k in range(ROWS_PER_TEC // CH):
        pltpu.sync_copy(rows0, agg_s.at[pl.ds(r_base + k * CH, CH), :])

    plsc.subcore_barrier()

    def _run(h_ref, w_ref):
        e_base = s * EPT
        g_base = eoff + e_base

        def _step(ci, p):
            q = 1 - p
            pltpu.make_async_copy(h_ref.at[idxj[p]], rows[p], sg[p]).wait()
            pltpu.make_async_copy(w_ref.at[pl.ds(0, CH), :], wij[p],
                                  sw[p]).wait()

            @pl.when(ci + 2 < NCHUNK)
            def _():
                g2 = g_base + (ci + 2) * CH
                pltpu.async_copy(idxj_hbm.at[pl.ds(g2, CH)], idxj[p], sj[p])

            @pl.when(ci + 1 < NCHUNK)
            def _():
                e1 = e_base + (ci + 1) * CH
                pltpu.make_async_copy(idxj_hbm.at[pl.ds(0, CH)], idxj[q],
                                      sj[q]).wait()

                @pl.when(ci >= 1)
                def _():
                    pltpu.make_async_copy(rows[q], agg_s.at[idxi[q]],
                                          ssc[q]).wait()
                    g1 = g_base + (ci + 1) * CH
                    pltpu.async_copy(idxi_hbm.at[pl.ds(g1, CH)], idxi[q],
                                     sii[q])

                pltpu.async_copy(h_ref.at[idxj[q]], rows[q], sg[q])
                pltpu.async_copy(w_ref.at[pl.ds(e1, CH), :], wij[q], sw[q])

            def _mul(r, cc):
                for rr in range(2):
                    r2 = 2 * r + rr
                    for j in range(8):
                        sl = pl.ds(j * 16, 16)
                        rows[p][r2, sl] = rows[p][r2, sl] * wij[p][r2, sl]
                return cc
            lax.fori_loop(0, CH // 2, _mul, 0)

            pltpu.make_async_copy(idxi_hbm.at[pl.ds(0, CH)], idxi[p],
                                  sii[p]).wait()
            pltpu.async_copy(rows[p], agg_s.at[idxi[p]], ssc[p], add=True)

        pltpu.async_copy(idxi_hbm.at[pl.ds(g_base, CH)], idxi[0], sii[0])
        pltpu.async_copy(idxi_hbm.at[pl.ds(g_base + CH, CH)], idxi[1], sii[1])
        pltpu.sync_copy(idxj_hbm.at[pl.ds(g_base, CH)], idxj[0])
        pltpu.async_copy(idxj_hbm.at[pl.ds(g_base + CH, CH)], idxj[1], sj[1])
        pltpu.async_copy(h_ref.at[idxj[0]], rows[0], sg[0])
        pltpu.async_copy(w_ref.at[pl.ds(e_base, CH), :], wij[0], sw[0])

        def _pair(k, carry):
            _step(2 * k, 0)
            _step(2 * k + 1, 1)
            return carry
        lax.fori_loop(0, NCHUNK // 2, _pair, 0)

        pltpu.make_async_copy(rows[0], agg_s.at[idxi[0]], ssc[0]).wait()
        pltpu.make_async_copy(rows[1], agg_s.at[idxi[1]], ssc[1]).wait()

    @pl.when(c == 0)
    def _():
        _run(hlo, wlo)

    @pl.when(c == 1)
    def _():
        _run(hhi, whi)

    plsc.subcore_barrier()

    for k in range(ROWS_PER_TEC // CH):
        rr = r_base + k * CH
        pltpu.sync_copy(agg_s.at[pl.ds(rr, CH), :], rows0)

        @pl.when(c == 0)
        def _():
            pltpu.sync_copy(rows0, out_lo.at[pl.ds(rr, CH), :])

        @pl.when(c == 1)
        def _():
            pltpu.sync_copy(rows0, out_hi.at[pl.ds(rr, CH), :])


@functools.cache
def _make_cfconv_sc(eoff):
    return functools.partial(
        pl.kernel,
        mesh=plsc.VectorSubcoreMesh(core_axis_name="c", subcore_axis_name="s"),
        out_type=(jax.ShapeDtypeStruct((NPAD, H), jnp.float32),
                  jax.ShapeDtypeStruct((NPAD, H), jnp.float32)),
        scratch_types=[
            pltpu.VMEM((CH,), jnp.int32),
            pltpu.VMEM((CH,), jnp.int32),
            pltpu.VMEM((CH,), jnp.int32),
            pltpu.VMEM((CH,), jnp.int32),
            pltpu.VMEM((CH, H), jnp.float32),
            pltpu.VMEM((CH, H), jnp.float32),
            pltpu.VMEM((CH, H), jnp.float32),
            pltpu.VMEM((CH, H), jnp.float32),
            pltpu.VMEM_SHARED((NPAD, H), jnp.float32),
            pltpu.SemaphoreType.DMA,
            pltpu.SemaphoreType.DMA,
            pltpu.SemaphoreType.DMA,
            pltpu.SemaphoreType.DMA,
            pltpu.SemaphoreType.DMA,
            pltpu.SemaphoreType.DMA,
            pltpu.SemaphoreType.DMA,
            pltpu.SemaphoreType.DMA,
            pltpu.SemaphoreType.DMA,
            pltpu.SemaphoreType.DMA,
        ],
    )(functools.partial(_cfconv_sc_body, eoff))


def kernel(x, f_ij, idx_i, idx_j, rcut_ij,
           W_in2f, Wf1, bf1, Wf2, bf2, Wo1, bo1, Wo2, bo2):
    h_lo, h_hi = pl.pallas_call(
        _in2f_body,
        grid=(10,),
        in_specs=[
            pl.BlockSpec((N // 10, D), lambda i: (i, 0)),
            pl.BlockSpec((D, D), lambda i: (0, 0)),
        ],
        out_specs=[
            pl.BlockSpec((N // 10, H), lambda i: (i, 0)),
            pl.BlockSpec((N // 10, H), lambda i: (i, 0)),
        ],
        out_shape=[
            jax.ShapeDtypeStruct((N, H), jnp.float32),
            jax.ShapeDtypeStruct((N, H), jnp.float32),
        ],
    )(x, W_in2f)

    pad = E_PAD - E
    f_pad = jnp.pad(f_ij, ((0, pad), (0, 0)))
    rc_pad = jnp.pad(rcut_ij, (0, pad))[:, None]
    idxj32 = jnp.pad(idx_j.astype(jnp.int32), (0, pad))
    idxi32 = jnp.pad(idx_i.astype(jnp.int32), (0, pad))

    BE = 512

    def _filter_half(block_off):
        return pl.pallas_call(
            _filter_body,
            grid=(E_HALF // BE,),
            in_specs=[
                pl.BlockSpec((BE, R), lambda i: (i + block_off, 0)),
                pl.BlockSpec((BE, 1), lambda i: (i + block_off, 0)),
                pl.BlockSpec((R, D), lambda i: (0, 0)),
                pl.BlockSpec((1, D), lambda i: (0, 0)),
                pl.BlockSpec((D, D), lambda i: (0, 0)),
                pl.BlockSpec((1, D), lambda i: (0, 0)),
            ],
            out_specs=[
                pl.BlockSpec((BE, H), lambda i: (i, 0)),
                pl.BlockSpec((BE, H), lambda i: (i, 0)),
            ],
            out_shape=[
                jax.ShapeDtypeStruct((E_HALF, H), jnp.float32),
                jax.ShapeDtypeStruct((E_HALF, H), jnp.float32),
            ],
        )(f_pad, rc_pad, Wf1, bf1[None, :], Wf2, bf2[None, :])

    wa_lo, wa_hi = _filter_half(0)
    agg_a_lo, agg_a_hi = _make_cfconv_sc(0)(
        h_lo, h_hi, wa_lo, wa_hi, idxj32, idxi32)
    wb_lo, wb_hi = _filter_half(E_HALF // BE)
    agg_b_lo, agg_b_hi = _make_cfconv_sc(E_HALF)(
        h_lo, h_hi, wb_lo, wb_hi, idxj32, idxi32)

    BN = 1024
    out = pl.pallas_call(
        _out_body,
        grid=(NPAD // BN,),
        in_specs=[
            pl.BlockSpec((BN, H), lambda i: (i, 0)),
            pl.BlockSpec((BN, H), lambda i: (i, 0)),
            pl.BlockSpec((BN, H), lambda i: (i, 0)),
            pl.BlockSpec((BN, H), lambda i: (i, 0)),
            pl.BlockSpec((D, D), lambda i: (0, 0)),
            pl.BlockSpec((1, D), lambda i: (0, 0)),
            pl.BlockSpec((D, D), lambda i: (0, 0)),
            pl.BlockSpec((1, D), lambda i: (0, 0)),
        ],
        out_specs=pl.BlockSpec((BN, D), lambda i: (i, 0)),
        out_shape=jax.ShapeDtypeStruct((NPAD, D), jnp.float32),
    )(agg_a_lo, agg_a_hi, agg_b_lo, agg_b_hi,
      Wo1, bo1[None, :], Wo2, bo2[None, :])

    return out[:N]
